# baseline (speedup 1.0000x reference)
import functools

import jax
import jax.numpy as jnp
from jax import lax
from jax.experimental import pallas as pl
from jax.experimental.pallas import tpu as pltpu
from jax.experimental.pallas import tpu_sc as plsc

VOCAB_SIZE = 1000000
EMBED_SIZE = 32
SEQ = 50
BATCH = 16384

_info = plsc.get_sparse_core_info()
_NC, _NS = _info.num_cores, _info.num_subcores
_NW = _NC * _NS
_BW = BATCH // _NW
_TPW = _BW // 128


def _make_kernel():
  mesh = plsc.VectorSubcoreMesh(core_axis_name="c", subcore_axis_name="s")

  @functools.partial(
      pl.kernel,
      out_type=jax.ShapeDtypeStruct((SEQ, 4, BATCH // 128, 8, 128),
                                    jnp.float32),
      mesh=mesh,
      scratch_types=[
          pltpu.VMEM((SEQ, _BW), jnp.int32),
          pltpu.VMEM((_BW, EMBED_SIZE), jnp.float32),
          pltpu.VMEM((_BW, EMBED_SIZE), jnp.float32),
          pltpu.VMEM((4, _TPW, 8, 128), jnp.float32),
          pltpu.VMEM((4, _TPW, 8, 128), jnp.float32),
          pltpu.SemaphoreType.DMA,
          pltpu.SemaphoreType.DMA,
          pltpu.SemaphoreType.DMA,
          pltpu.SemaphoreType.DMA,
      ],
      compiler_params=pltpu.CompilerParams(use_tc_tiling_on_sc=False,
                                           needs_layout_passes=False),
  )
  def embed(xt_hbm, table_hbm, out_hbm, idx_v, rows0, rows1, ob0, ob1,
            g0, g1, w0, w1):
    wid = lax.axis_index("s") * _NC + lax.axis_index("c")
    b0 = wid * _BW
    btg0 = wid * _TPW
    rows = (rows0, rows1)
    obuf = (ob0, ob1)
    gsem = (g0, g1)
    wsem = (w0, w1)
    iota16 = lax.iota(jnp.int32, 16)

    pltpu.sync_copy(xt_hbm.at[:, pl.ds(b0, _BW)], idx_v)

    pltpu.async_copy(table_hbm.at[idx_v.at[0]], rows0, g0)
    pltpu.async_copy(table_hbm.at[idx_v.at[1]], rows1, g1)

    def outer(t):
      for b in range(2):
        s = 2 * t + b
        pltpu.make_async_copy(table_hbm.at[pl.ds(0, _BW)], rows[b],
                              gsem[b]).wait()

        @pl.when(t > 0)
        def _():
          pltpu.make_async_copy(obuf[b],
                                out_hbm.at[s, :, pl.ds(btg0, _TPW)],
                                wsem[b]).wait()

        for bt in range(_TPW):
          row_vecs = [iota16 + (bt * 128 + g * 16) for g in range(8)]

          @plsc.parallel_loop(0, EMBED_SIZE, unroll=4)
          def transpose(d):
            dcol = jnp.full((16,), d, jnp.int32)
            dt = d >> 3
            dr = d & 7
            for g in range(8):
              vals = plsc.load_gather(rows[b], [row_vecs[g], dcol])
              obuf[b][dt, bt, dr, pl.ds(g * 16, 16)] = vals

        pltpu.async_copy(obuf[b], out_hbm.at[s, :, pl.ds(btg0, _TPW)],
                         wsem[b])

        @pl.when(s + 2 < SEQ)
        def _():
          pltpu.async_copy(table_hbm.at[idx_v.at[s + 2]], rows[b], gsem[b])

    pl.loop(0, SEQ // 2)(outer)

    for b in range(2):
      pltpu.make_async_copy(obuf[b], out_hbm.at[SEQ - 2 + b, :,
                                                pl.ds(btg0, _TPW)],
                            wsem[b]).wait()

  return embed


_embed = _make_kernel()


@jax.jit
def kernel(x, table):
  xt = jnp.transpose(x).astype(jnp.int32)
  out5d = _embed(xt, table)
  return out5d.transpose(2, 4, 0, 1, 3).reshape(BATCH, SEQ, EMBED_SIZE)


# speedup vs baseline: 1.4283x; 1.4283x over previous
import functools

import jax
import jax.numpy as jnp
from jax import lax
from jax.experimental import pallas as pl
from jax.experimental.pallas import tpu as pltpu
from jax.experimental.pallas import tpu_sc as plsc

VOCAB_SIZE = 1000000
EMBED_SIZE = 32
SEQ = 50
BATCH = 16384

_info = plsc.get_sparse_core_info()
_NC, _NS = _info.num_cores, _info.num_subcores
_NW = _NC * _NS
_BW = BATCH // _NW
_TPW = _BW // 128
_PAD = 129


def _make_kernel():
  mesh = plsc.VectorSubcoreMesh(core_axis_name="c", subcore_axis_name="s")

  @functools.partial(
      pl.kernel,
      out_type=jax.ShapeDtypeStruct((SEQ, 4 * (BATCH // 128) * 8, 128),
                                    jnp.float32),
      mesh=mesh,
      scratch_types=[
          pltpu.VMEM((SEQ, _BW), jnp.int32),
          pltpu.VMEM((_BW, EMBED_SIZE), jnp.float32),
          pltpu.VMEM((_BW, EMBED_SIZE), jnp.float32),
          pltpu.VMEM((_TPW * 32, _PAD), jnp.float32),
          pltpu.VMEM((_TPW * 32, _PAD), jnp.float32),
          pltpu.SemaphoreType.DMA,
          pltpu.SemaphoreType.DMA,
          pltpu.SemaphoreType.DMA,
          pltpu.SemaphoreType.DMA,
      ],
      compiler_params=pltpu.CompilerParams(use_tc_tiling_on_sc=False,
                                           needs_layout_passes=False),
  )
  def embed(xt_hbm, table_hbm, out_hbm, idx_v, rows0, rows1, ob0, ob1,
            g0, g1, w0, w1):
    wid = lax.axis_index("s") * _NC + lax.axis_index("c")
    b0 = wid * _BW
    btg0 = wid * _TPW
    rows = (rows0, rows1)
    obuf = (ob0, ob1)
    gsem = (g0, g1)
    wsem = (w0, w1)
    iota16 = lax.iota(jnp.int32, 16)

    pltpu.sync_copy(xt_hbm.at[:, pl.ds(b0, _BW)], idx_v)

    pltpu.async_copy(table_hbm.at[idx_v.at[0]], rows0, g0)
    pltpu.async_copy(table_hbm.at[idx_v.at[1]], rows1, g1)

    def wait_bytes_64k(sem):
      pltpu.make_async_copy(table_hbm.at[pl.ds(0, _BW)], rows0, sem).wait()

    def outer(t):
      for b in range(2):
        s = 2 * t + b
        wait_bytes_64k(gsem[b])

        @pl.when(t > 0)
        def _():
          wait_bytes_64k(wsem[b])

        for bt in range(_TPW):
          rv = (iota16 + bt * 32, iota16 + (bt * 32 + 16))

          @plsc.parallel_loop(0, 128, unroll=4)
          def transpose(i):
            vals0 = rows[b][bt * 128 + i, pl.ds(0, 16)]
            vals1 = rows[b][bt * 128 + i, pl.ds(16, 16)]
            col = jnp.full((16,), i, jnp.int32)
            plsc.store_scatter(obuf[b], [rv[0], col], vals0)
            plsc.store_scatter(obuf[b], [rv[1], col], vals1)

        for dt in range(4):
          for bt in range(_TPW):
            pltpu.async_copy(
                obuf[b].at[pl.ds(bt * 32 + dt * 8, 8), pl.ds(0, 128)],
                out_hbm.at[s, pl.ds(dt * (BATCH // 128) * 8
                                    + (btg0 + bt) * 8, 8)],
                wsem[b])

        @pl.when(s + 2 < SEQ)
        def _():
          pltpu.async_copy(table_hbm.at[idx_v.at[s + 2]], rows[b], gsem[b])

    pl.loop(0, SEQ // 2)(outer)

    wait_bytes_64k(w0)
    wait_bytes_64k(w1)

  return embed


_embed = _make_kernel()


@jax.jit
def kernel(x, table):
  xt = jnp.transpose(x).astype(jnp.int32)
  out3d = _embed(xt, table)
  out5d = out3d.reshape(SEQ, 4, BATCH // 128, 8, 128)
  return out5d.transpose(2, 4, 0, 1, 3).reshape(BATCH, SEQ, EMBED_SIZE)


# speedup vs baseline: 1.4289x; 1.0004x over previous
import functools

import jax
import jax.numpy as jnp
from jax import lax
from jax.experimental import pallas as pl
from jax.experimental.pallas import tpu as pltpu
from jax.experimental.pallas import tpu_sc as plsc

VOCAB_SIZE = 1000000
EMBED_SIZE = 32
SEQ = 50
BATCH = 16384

_info = plsc.get_sparse_core_info()
_NC, _NS = _info.num_cores, _info.num_subcores
_NW = _NC * _NS
_BW = BATCH // _NW
_TPW = _BW // 128
_PAD = 129


def _make_kernel():
  mesh = plsc.VectorSubcoreMesh(core_axis_name="c", subcore_axis_name="s")

  @functools.partial(
      pl.kernel,
      out_type=jax.ShapeDtypeStruct((SEQ, 4 * (BATCH // 128) * 8, 128),
                                    jnp.float32),
      mesh=mesh,
      scratch_types=[
          pltpu.VMEM((SEQ, _BW), jnp.int32),
          pltpu.VMEM((_BW, EMBED_SIZE), jnp.float32),
          pltpu.VMEM((_BW, EMBED_SIZE), jnp.float32),
          pltpu.VMEM((_TPW * 32, _PAD), jnp.float32),
          pltpu.VMEM((_TPW * 32, _PAD), jnp.float32),
          pltpu.SemaphoreType.DMA,
          pltpu.SemaphoreType.DMA,
          pltpu.SemaphoreType.DMA,
          pltpu.SemaphoreType.DMA,
      ],
      compiler_params=pltpu.CompilerParams(use_tc_tiling_on_sc=False,
                                           needs_layout_passes=False,
                                           disable_bounds_checks=True,
                                           disable_semaphore_checks=True,
                                           skip_device_barrier=True),
  )
  def embed(xt_hbm, table_hbm, out_hbm, idx_v, rows0, rows1, ob0, ob1,
            g0, g1, w0, w1):
    wid = lax.axis_index("s") * _NC + lax.axis_index("c")
    b0 = wid * _BW
    btg0 = wid * _TPW
    rows = (rows0, rows1)
    obuf = (ob0, ob1)
    gsem = (g0, g1)
    wsem = (w0, w1)
    iota16 = lax.iota(jnp.int32, 16)

    pltpu.sync_copy(xt_hbm.at[:, pl.ds(b0, _BW)], idx_v)

    pltpu.async_copy(table_hbm.at[idx_v.at[0]], rows0, g0)
    pltpu.async_copy(table_hbm.at[idx_v.at[1]], rows1, g1)

    def wait_bytes_64k(sem):
      pltpu.make_async_copy(table_hbm.at[pl.ds(0, _BW)], rows0, sem).wait()

    def outer(t):
      for b in range(2):
        s = 2 * t + b
        wait_bytes_64k(gsem[b])

        @pl.when(t > 0)
        def _():
          wait_bytes_64k(wsem[b])

        for bt in range(_TPW):
          rv = (iota16 + bt * 32, iota16 + (bt * 32 + 16))

          @plsc.parallel_loop(0, 128, unroll=4)
          def transpose(i):
            vals0 = rows[b][bt * 128 + i, pl.ds(0, 16)]
            vals1 = rows[b][bt * 128 + i, pl.ds(16, 16)]
            col = jnp.full((16,), i, jnp.int32)
            plsc.store_scatter(obuf[b], [rv[0], col], vals0)
            plsc.store_scatter(obuf[b], [rv[1], col], vals1)

        for dt in range(4):
          for bt in range(_TPW):
            pltpu.async_copy(
                obuf[b].at[pl.ds(bt * 32 + dt * 8, 8), pl.ds(0, 128)],
                out_hbm.at[s, pl.ds(dt * (BATCH // 128) * 8
                                    + (btg0 + bt) * 8, 8)],
                wsem[b])

        @pl.when(s + 2 < SEQ)
        def _():
          pltpu.async_copy(table_hbm.at[idx_v.at[s + 2]], rows[b], gsem[b])

    pl.loop(0, SEQ // 2)(outer)

    wait_bytes_64k(w0)
    wait_bytes_64k(w1)

  return embed


_embed = _make_kernel()


@jax.jit
def kernel(x, table):
  xt = jnp.transpose(x).astype(jnp.int32)
  out3d = _embed(xt, table)
  out5d = out3d.reshape(SEQ, 4, BATCH // 128, 8, 128)
  return out5d.transpose(2, 4, 0, 1, 3).reshape(BATCH, SEQ, EMBED_SIZE)


# speedup vs baseline: 1.4339x; 1.0035x over previous
import functools

import jax
import jax.numpy as jnp
from jax import lax
from jax.experimental import pallas as pl
from jax.experimental.pallas import tpu as pltpu
from jax.experimental.pallas import tpu_sc as plsc

VOCAB_SIZE = 1000000
EMBED_SIZE = 32
SEQ = 50
BATCH = 16384

_info = plsc.get_sparse_core_info()
_NC, _NS = _info.num_cores, _info.num_subcores
_NW = _NC * _NS
_BW = BATCH // _NW
_TPW = _BW // 128
_PAD = 129


def _make_kernel():
  mesh = plsc.VectorSubcoreMesh(core_axis_name="c", subcore_axis_name="s")

  @functools.partial(
      pl.kernel,
      out_type=jax.ShapeDtypeStruct((SEQ, 4 * (BATCH // 128) * 8, 128),
                                    jnp.float32),
      mesh=mesh,
      scratch_types=[
          pltpu.VMEM((SEQ, _BW), jnp.int32),
          pltpu.VMEM((_BW, EMBED_SIZE), jnp.float32),
          pltpu.VMEM((_BW, EMBED_SIZE), jnp.float32),
          pltpu.VMEM((_TPW * 32, _PAD), jnp.float32),
          pltpu.VMEM((_TPW * 32, _PAD), jnp.float32),
          pltpu.SemaphoreType.DMA,
          pltpu.SemaphoreType.DMA,
          pltpu.SemaphoreType.DMA,
          pltpu.SemaphoreType.DMA,
      ],
      compiler_params=pltpu.CompilerParams(use_tc_tiling_on_sc=False,
                                           needs_layout_passes=False),
  )
  def embed(xt_hbm, table_hbm, out_hbm, idx_v, rows0, rows1, ob0, ob1,
            g0, g1, w0, w1):
    wid = lax.axis_index("s") * _NC + lax.axis_index("c")
    b0 = wid * _BW
    btg0 = wid * _TPW
    rows = (rows0, rows1)
    obuf = (ob0, ob1)
    gsem = (g0, g1)
    wsem = (w0, w1)
    iota16 = lax.iota(jnp.int32, 16)

    pltpu.sync_copy(xt_hbm.at[:, pl.ds(b0, _BW)], idx_v)

    pltpu.async_copy(table_hbm.at[idx_v.at[0]], rows0, g0)
    pltpu.async_copy(table_hbm.at[idx_v.at[1]], rows1, g1)

    def wait_bytes_64k(sem):
      pltpu.make_async_copy(table_hbm.at[pl.ds(0, _BW)], rows0, sem).wait()

    def outer(t):
      for b in range(2):
        s = 2 * t + b
        wait_bytes_64k(gsem[b])

        @pl.when(t > 0)
        def _():
          wait_bytes_64k(wsem[b])

        for bt in range(_TPW):
          rv = (iota16 + bt * 32, iota16 + (bt * 32 + 16))

          @plsc.parallel_loop(0, 128, unroll=4)
          def transpose(i):
            vals0 = rows[b][bt * 128 + i, pl.ds(0, 16)]
            vals1 = rows[b][bt * 128 + i, pl.ds(16, 16)]
            col = jnp.full((16,), i, jnp.int32)
            plsc.store_scatter(obuf[b], [rv[0], col], vals0)
            plsc.store_scatter(obuf[b], [rv[1], col], vals1)

        for dt in range(4):
          for bt in range(_TPW):
            pltpu.async_copy(
                obuf[b].at[pl.ds(bt * 32 + dt * 8, 8), pl.ds(0, 128)],
                out_hbm.at[s, pl.ds(dt * (BATCH // 128) * 8
                                    + (btg0 + bt) * 8, 8)],
                wsem[b])

        @pl.when(s + 2 < SEQ)
        def _():
          pltpu.async_copy(table_hbm.at[idx_v.at[s + 2]], rows[b], gsem[b])

    pl.loop(0, SEQ // 2)(outer)

    wait_bytes_64k(w0)
    wait_bytes_64k(w1)

  return embed


_embed = _make_kernel()


@jax.jit
def kernel(x, table):
  xt = jnp.transpose(x).astype(jnp.int32)
  out3d = _embed(xt, table)
  out5d = out3d.reshape(SEQ, 4, BATCH // 128, 8, 128)
  return out5d.transpose(2, 4, 0, 1, 3).reshape(BATCH, SEQ, EMBED_SIZE)
